# baseline (speedup 1.0000x reference)
import jax
import jax.numpy as jnp
from jax.experimental import pallas as pl
from jax.experimental.pallas import tpu as pltpu

_DIST2_THRESH = 0.2 * 0.2
_CONF_THRESH = 0.7
_BIG = 1e30


_STRIP = 64


def _adj_kernel(in_ref, out_ref):
    bs, n = out_ref.shape[0], out_ref.shape[2]
    rows_local = jax.lax.broadcasted_iota(jnp.int32, (_STRIP, n), 0)
    cols = jax.lax.broadcasted_iota(jnp.int32, (_STRIP, n), 1)
    for st in range(n // _STRIP):
        s0 = st * _STRIP
        ne = (rows_local + s0) != cols
        for s in range(bs):
            x = in_ref[s, 0:1, :]
            y = in_ref[s, 1:2, :]
            z = in_ref[s, 2:3, :]
            t_row = in_ref[s, 3:4, :]
            xc = jnp.transpose(in_ref[s, 0:1, s0 : s0 + _STRIP])
            yc = jnp.transpose(in_ref[s, 1:2, s0 : s0 + _STRIP])
            zc = jnp.transpose(in_ref[s, 2:3, s0 : s0 + _STRIP])
            tc = jnp.transpose(in_ref[s, 4:5, s0 : s0 + _STRIP])
            dx = xc - x
            dy = yc - y
            dz = zc - z
            d2 = dx * dx + dy * dy + dz * dz
            t = tc + t_row
            out_ref[s, s0 : s0 + _STRIP, :] = ((d2 > t) & ne).astype(
                jnp.float32
            )


def kernel(centroid, obj_conf):
    B, N, _ = centroid.shape
    conf_ok = obj_conf >= _CONF_THRESH
    half = jnp.full_like(obj_conf, 0.5 * _DIST2_THRESH)
    t_row = jnp.where(conf_ok, half, jnp.inf)
    dist_enabled = (jnp.arange(B)[:, None] >= 2) & (jnp.arange(N)[None, :] >= 2)
    t_col = jnp.where(conf_ok, jnp.where(dist_enabled, half, -_BIG), jnp.inf)
    packed = jnp.concatenate(
        [
            jnp.transpose(centroid, (0, 2, 1)),
            t_row[:, None, :],
            t_col[:, None, :],
        ],
        axis=1,
    )
    return pl.pallas_call(
        _adj_kernel,
        grid=(B // 4,),
        in_specs=[pl.BlockSpec((4, 5, N), lambda b: (b, 0, 0))],
        out_specs=pl.BlockSpec((4, N, N), lambda b: (b, 0, 0)),
        out_shape=jax.ShapeDtypeStruct((B, N, N), jnp.float32),
    )(packed)


# speedup vs baseline: 1.2094x; 1.2094x over previous
import jax
import jax.numpy as jnp
from jax.experimental import pallas as pl
from jax.experimental.pallas import tpu as pltpu

_DIST2_THRESH = 0.2 * 0.2
_CONF_THRESH = 0.7
_BIG = 1e30
_BS = 4


def _adj_kernel(in_ref, out_ref, odiag_ref):
    n = out_ref.shape[2]

    @pl.when(pl.program_id(0) == 0)
    def _init():
        rows = jax.lax.broadcasted_iota(jnp.int32, (n, n), 0)
        cols = jax.lax.broadcasted_iota(jnp.int32, (n, n), 1)
        odiag_ref[...] = (rows != cols).astype(jnp.float32)

    od = odiag_ref[...]
    for s in range(_BS):
        x = in_ref[s, 0:1, :]
        y = in_ref[s, 1:2, :]
        z = in_ref[s, 2:3, :]
        t_row = in_ref[s, 3:4, :]
        xc = jnp.transpose(in_ref[s, 0:1, :])
        yc = jnp.transpose(in_ref[s, 1:2, :])
        zc = jnp.transpose(in_ref[s, 2:3, :])
        tc = jnp.transpose(in_ref[s, 4:5, :])
        g = xc * x + yc * y + zc * z
        t = tc + t_row
        out_ref[s] = jnp.where(g < t, od, 0.0)


def kernel(centroid, obj_conf):
    B, N, _ = centroid.shape
    n2 = jnp.sum(centroid * centroid, axis=-1)
    conf_ok = obj_conf >= _CONF_THRESH
    half = (n2 - 0.5 * _DIST2_THRESH) * 0.5
    t_row = jnp.where(conf_ok, half, -jnp.inf)
    dist_enabled = (jnp.arange(B)[:, None] >= 2) & (jnp.arange(N)[None, :] >= 2)
    t_col = jnp.where(conf_ok, jnp.where(dist_enabled, half, _BIG), -jnp.inf)
    packed = jnp.concatenate(
        [
            jnp.transpose(centroid, (0, 2, 1)),
            t_row[:, None, :],
            t_col[:, None, :],
        ],
        axis=1,
    )
    return pl.pallas_call(
        _adj_kernel,
        grid=(B // _BS,),
        in_specs=[pl.BlockSpec((_BS, 5, N), lambda b: (b, 0, 0))],
        out_specs=pl.BlockSpec((_BS, N, N), lambda b: (b, 0, 0)),
        out_shape=jax.ShapeDtypeStruct((B, N, N), jnp.float32),
        scratch_shapes=[pltpu.VMEM((N, N), jnp.float32)],
    )(packed)
